# baseline (speedup 1.0000x reference)
import functools

import jax
import jax.numpy as jnp
from jax import lax
from jax.experimental import pallas as pl
from jax.experimental.pallas import tpu as pltpu
from jax.experimental.pallas import tpu_sc as plsc

N_TOK = 10
_info = plsc.get_sparse_core_info()
_NC = _info.num_cores
_NS = _info.num_subcores
_NW = _NC * _NS


def _make_gather(nb, seq, d, t):
  mesh = plsc.VectorSubcoreMesh(core_axis_name="c", subcore_axis_name="s")
  iters = (nb + _NW - 1) // _NW
  g0 = 128
  g1 = seq - g0

  @functools.partial(
      pl.kernel,
      mesh=mesh,
      out_type=jax.ShapeDtypeStruct((nb * seq, d), jnp.float32),
      scratch_types=[
          pltpu.VMEM((seq,), jnp.int32),
          pltpu.VMEM((seq,), jnp.int32),
          pltpu.VMEM((N_TOK + seq, d), jnp.float32),
          pltpu.VMEM((N_TOK + seq, d), jnp.float32),
          pltpu.SemaphoreType.DMA,
          pltpu.SemaphoreType.DMA,
          pltpu.SemaphoreType.DMA,
          pltpu.SemaphoreType.DMA,
          pltpu.SemaphoreType.DMA,
          pltpu.SemaphoreType.DMA,
      ],
  )
  def k(idx_hbm, table_hbm, learned_hbm, out_hbm, idx_v0, idx_v1, rows_v0,
        rows_v1, si0, si1, sg0, sg1, so0, so1):
    wid = lax.axis_index("s") * _NC + lax.axis_index("c")
    idx_v = (idx_v0, idx_v1)
    rows_v = (rows_v0, rows_v1)
    sem_i = (si0, si1)
    sem_g = (sg0, sg1)
    sem_o = (so0, so1)

    def b_of(i):
      raw = wid + i * _NW
      return jnp.where(raw < nb, raw, wid)

    def idx_copy(p, b):
      return pltpu.make_async_copy(
          idx_hbm.at[pl.ds(b * seq, seq)], idx_v[p], sem_i[p])

    def gather_copies(p):
      c0 = pltpu.make_async_copy(
          table_hbm.at[idx_v[p].at[pl.ds(0, g0)]],
          rows_v[p].at[pl.ds(N_TOK, g0)], sem_g[p])
      c1 = pltpu.make_async_copy(
          table_hbm.at[idx_v[p].at[pl.ds(g0, g1)]],
          rows_v[p].at[pl.ds(N_TOK + g0, g1)], sem_g[p])
      return c0, c1

    def out_copy(p, b):
      off = jnp.where(b >= t, 0, N_TOK)
      return pltpu.make_async_copy(
          rows_v[p].at[pl.ds(off, seq)],
          out_hbm.at[pl.ds(b * seq, seq)], sem_o[p])

    def refresh(p, b):
      @pl.when((b >= t) & (b < t + 2 * _NW))
      def _():
        pltpu.sync_copy(
            learned_hbm.at[pl.ds(0, 16)], rows_v[p].at[pl.ds(0, 16)])

      @pl.when((b >= 2 * t) & (b < 2 * t + 2 * _NW))
      def _():
        pltpu.sync_copy(
            learned_hbm.at[pl.ds(16, 16)], rows_v[p].at[pl.ds(0, 16)])

    def fire_gather(p, b):
      refresh(p, b)
      c0, c1 = gather_copies(p)
      c0.start()
      c1.start()

    def drain_gather(p):
      c0, c1 = gather_copies(p)
      c0.wait()
      c1.wait()

    def step(i, p):
      q = 1 - p
      b = b_of(i)
      out_copy(p, b_of(i - 2)).wait()
      idx_copy(p, b).wait()
      fire_gather(p, b)
      drain_gather(q)
      out_copy(q, b_of(i - 1)).start()

      @pl.when(i + 1 < iters)
      def _():
        idx_copy(q, b_of(i + 1)).start()

    idx_copy(0, b_of(0)).start()
    idx_copy(0, b_of(0)).wait()
    fire_gather(0, b_of(0))
    idx_copy(1, b_of(1)).start()
    idx_copy(1, b_of(1)).wait()
    fire_gather(1, b_of(1))
    drain_gather(0)
    out_copy(0, b_of(0)).start()
    idx_copy(0, b_of(2)).start()

    def body(j, _):
      step(2 * j, 0)
      step(2 * j + 1, 1)
      return 0

    lax.fori_loop(1, iters // 2, body, 0)

    p_last = (iters - 1) % 2
    drain_gather(p_last)
    out_copy(p_last, b_of(iters - 1)).start()
    out_copy(1 - p_last, b_of(iters - 2)).wait()
    out_copy(p_last, b_of(iters - 1)).wait()

  return k


def kernel(bert_indices_add, wte, learned_embedding_1, learned_embedding_2):
  B, S = bert_indices_add.shape
  t = B // 3
  nb = 3 * t
  n_tok = learned_embedding_1.shape[0]
  d = wte.shape[1]
  idx = bert_indices_add.astype(jnp.int32)
  shifted = jnp.pad(idx[t:nb, : S - n_tok], ((0, 0), (0, n_tok)))
  flat_idx = jnp.concatenate([idx[:t], shifted], axis=0).reshape(-1)
  pad = jnp.zeros((16 - n_tok, d), jnp.float32)
  learned = jnp.concatenate(
      [learned_embedding_1, pad, learned_embedding_2, pad], axis=0)
  out = _make_gather(nb, S, d, t)(flat_idx, wte, learned)
  return out.reshape(nb, S, d)


# speedup vs baseline: 1.0004x; 1.0004x over previous
import functools

import jax
import jax.numpy as jnp
from jax import lax
from jax.experimental import pallas as pl
from jax.experimental.pallas import tpu as pltpu
from jax.experimental.pallas import tpu_sc as plsc

N_TOK = 10
_info = plsc.get_sparse_core_info()
_NC = _info.num_cores
_NS = _info.num_subcores
_NW = _NC * _NS


def _make_gather(nb, seq, d, t):
  mesh = plsc.VectorSubcoreMesh(core_axis_name="c", subcore_axis_name="s")
  iters = (nb + _NW - 1) // _NW
  g0 = 128
  g1 = seq - g0

  @functools.partial(
      pl.kernel,
      mesh=mesh,
      out_type=jax.ShapeDtypeStruct((nb * seq, d), jnp.float32),
      scratch_types=[
          pltpu.VMEM((seq,), jnp.int32),
          pltpu.VMEM((seq,), jnp.int32),
          pltpu.VMEM((seq,), jnp.int32),
          pltpu.VMEM((N_TOK + seq, d), jnp.float32),
          pltpu.VMEM((N_TOK + seq, d), jnp.float32),
          pltpu.VMEM((N_TOK + seq, d), jnp.float32),
          pltpu.SemaphoreType.DMA,
          pltpu.SemaphoreType.DMA,
          pltpu.SemaphoreType.DMA,
          pltpu.SemaphoreType.DMA,
          pltpu.SemaphoreType.DMA,
          pltpu.SemaphoreType.DMA,
          pltpu.SemaphoreType.DMA,
          pltpu.SemaphoreType.DMA,
          pltpu.SemaphoreType.DMA,
      ],
  )
  def k(idx_hbm, table_hbm, learned_hbm, out_hbm, ix0, ix1, ix2, rv0, rv1,
        rv2, si0, si1, si2, sg0, sg1, sg2, so0, so1, so2):
    wid = lax.axis_index("s") * _NC + lax.axis_index("c")
    idx_v = (ix0, ix1, ix2)
    rows_v = (rv0, rv1, rv2)
    sem_i = (si0, si1, si2)
    sem_g = (sg0, sg1, sg2)
    sem_o = (so0, so1, so2)

    def b_of(i):
      raw = wid + i * _NW
      return jnp.where(raw < nb, raw, wid)

    def idx_copy(c, b):
      return pltpu.make_async_copy(
          idx_hbm.at[pl.ds(b * seq, seq)], idx_v[c], sem_i[c])

    def gather_copies(c):
      c0 = pltpu.make_async_copy(
          table_hbm.at[idx_v[c].at[pl.ds(0, g0)]],
          rows_v[c].at[pl.ds(N_TOK, g0)], sem_g[c])
      c1 = pltpu.make_async_copy(
          table_hbm.at[idx_v[c].at[pl.ds(g0, g1)]],
          rows_v[c].at[pl.ds(N_TOK + g0, g1)], sem_g[c])
      return c0, c1

    def out_copy(c, b):
      off = jnp.where(b >= t, 0, N_TOK)
      return pltpu.make_async_copy(
          rows_v[c].at[pl.ds(off, seq)],
          out_hbm.at[pl.ds(b * seq, seq)], sem_o[c])

    def refresh(c, b):
      @pl.when((b >= t) & (b < t + 3 * _NW))
      def _():
        pltpu.sync_copy(
            learned_hbm.at[pl.ds(0, 16)], rows_v[c].at[pl.ds(0, 16)])

      @pl.when((b >= 2 * t) & (b < 2 * t + 3 * _NW))
      def _():
        pltpu.sync_copy(
            learned_hbm.at[pl.ds(16, 16)], rows_v[c].at[pl.ds(0, 16)])

    def fire_gather(c, b):
      refresh(c, b)
      x0, x1 = gather_copies(c)
      x0.start()
      x1.start()

    def drain_gather(c):
      x0, x1 = gather_copies(c)
      x0.wait()
      x1.wait()

    def step(i, c, prime=False):
      cp = (c + 2) % 3
      b = b_of(i)
      if not prime:
        out_copy(c, b_of(i - 2)).wait()
      idx_copy(c, b).wait()
      fire_gather(c, b)
      if not prime:
        drain_gather(cp)
        out_copy(cp, b_of(i - 1)).start()

      @pl.when(i + 1 < iters)
      def _():
        idx_copy((c + 1) % 3, b_of(i + 1)).start()

    idx_copy(0, b_of(0)).start()
    idx_copy(0, b_of(0)).wait()
    fire_gather(0, b_of(0))
    idx_copy(1, b_of(1)).start()

    idx_copy(1, b_of(1)).wait()
    fire_gather(1, b_of(1))
    drain_gather(0)
    out_copy(0, b_of(0)).start()
    idx_copy(2, b_of(2)).start()

    idx_copy(2, b_of(2)).wait()
    fire_gather(2, b_of(2))
    drain_gather(1)
    out_copy(1, b_of(1)).start()
    idx_copy(0, b_of(3)).start()

    def body(m, _):
      i0 = 3 + 3 * m
      step(i0, 0)
      step(i0 + 1, 1)
      step(i0 + 2, 2)
      return 0

    n_tri = (iters - 3) // 3
    lax.fori_loop(0, n_tri, body, 0)
    for i in range(3 + 3 * n_tri, iters):
      step(i, i % 3)

    c_last = (iters - 1) % 3
    drain_gather(c_last)
    out_copy(c_last, b_of(iters - 1)).start()
    out_copy(0, b_of(iters - 1)).wait()
    out_copy(1, b_of(iters - 1)).wait()
    out_copy(2, b_of(iters - 1)).wait()

  return k


def kernel(bert_indices_add, wte, learned_embedding_1, learned_embedding_2):
  B, S = bert_indices_add.shape
  t = B // 3
  nb = 3 * t
  n_tok = learned_embedding_1.shape[0]
  d = wte.shape[1]
  idx = bert_indices_add.astype(jnp.int32)
  shifted = jnp.pad(idx[t:nb, : S - n_tok], ((0, 0), (0, n_tok)))
  flat_idx = jnp.concatenate([idx[:t], shifted], axis=0).reshape(-1)
  pad = jnp.zeros((16 - n_tok, d), jnp.float32)
  learned = jnp.concatenate(
      [learned_embedding_1, pad, learned_embedding_2, pad], axis=0)
  out = _make_gather(nb, S, d, t)(flat_idx, wte, learned)
  return out.reshape(nb, S, d)


# speedup vs baseline: 3.7478x; 3.7462x over previous
import functools

import jax
import jax.numpy as jnp
from jax import lax
from jax.experimental import pallas as pl
from jax.experimental.pallas import tpu as pltpu
from jax.experimental.pallas import tpu_sc as plsc

N_TOK = 10
_info = plsc.get_sparse_core_info()
_NC = _info.num_cores
_NS = _info.num_subcores
_NW = _NC * _NS


def _make_gather(nb, seq, d, t):
  mesh = plsc.VectorSubcoreMesh(core_axis_name="c", subcore_axis_name="s")
  iters = (nb + _NW - 1) // _NW
  g0 = 128
  g1 = seq - g0
  g1p = seq - N_TOK - g0

  @functools.partial(
      pl.kernel,
      mesh=mesh,
      out_type=jax.ShapeDtypeStruct((nb * seq, d), jnp.float32),
      scratch_types=[
          pltpu.VMEM((seq,), jnp.int32),
          pltpu.VMEM((seq,), jnp.int32),
          pltpu.VMEM((N_TOK + seq, d), jnp.float32),
          pltpu.VMEM((N_TOK + seq, d), jnp.float32),
          pltpu.SemaphoreType.DMA,
          pltpu.SemaphoreType.DMA,
          pltpu.SemaphoreType.DMA,
      ],
  )
  def k(idx_hbm, table_hbm, learned_hbm, out_hbm, ix0, ix1, rv0, rv1, sg,
        so0, so1):
    wid = lax.axis_index("s") * _NC + lax.axis_index("c")
    idx_v = (ix0, ix1)
    rows_v = (rv0, rv1)
    sem_o = (so0, so1)

    def b_of(i):
      raw = wid + i * _NW
      return jnp.where(raw < nb, raw, wid)

    def out_wait(p):
      pltpu.make_async_copy(
          rows_v[p].at[pl.ds(0, seq)],
          out_hbm.at[pl.ds(b_of(0) * seq, seq)], sem_o[p]).wait()

    def step(i, p, prime=False):
      b = b_of(i)
      base = b * seq
      pltpu.sync_copy(idx_hbm.at[pl.ds(base, seq)], idx_v[p])
      if not prime:
        out_wait(p)

      @pl.when((b >= t) & (b < t + 2 * _NW))
      def _():
        pltpu.sync_copy(
            learned_hbm.at[pl.ds(0, 16)], rows_v[p].at[pl.ds(0, 16)])

      @pl.when((b >= 2 * t) & (b < 2 * t + 2 * _NW))
      def _():
        pltpu.sync_copy(
            learned_hbm.at[pl.ds(16, 16)], rows_v[p].at[pl.ds(0, 16)])

      @pl.when(b < t)
      def _():
        c0 = pltpu.async_copy(
            table_hbm.at[idx_v[p].at[pl.ds(0, g0)]],
            rows_v[p].at[pl.ds(N_TOK, g0)], sg)
        c1 = pltpu.async_copy(
            table_hbm.at[idx_v[p].at[pl.ds(g0, g1)]],
            rows_v[p].at[pl.ds(N_TOK + g0, g1)], sg)
        c0.wait()
        c1.wait()
        pltpu.make_async_copy(
            rows_v[p].at[pl.ds(N_TOK, seq)],
            out_hbm.at[pl.ds(base, seq)], sem_o[p]).start()

      @pl.when(b >= t)
      def _():
        c0 = pltpu.async_copy(
            table_hbm.at[idx_v[p].at[pl.ds(0, g0)]],
            rows_v[p].at[pl.ds(N_TOK, g0)], sg)
        c1 = pltpu.async_copy(
            table_hbm.at[idx_v[p].at[pl.ds(g0, g1p)]],
            rows_v[p].at[pl.ds(N_TOK + g0, g1p)], sg)
        c0.wait()
        c1.wait()
        pltpu.make_async_copy(
            rows_v[p].at[pl.ds(0, seq)],
            out_hbm.at[pl.ds(base, seq)], sem_o[p]).start()

    step(0, 0, prime=True)
    step(1, 1, prime=True)

    def body(j, _):
      step(2 * j, 0)
      step(2 * j + 1, 1)
      return 0

    lax.fori_loop(1, iters // 2, body, 0)

    out_wait(0)
    out_wait(1)

  return k


def kernel(bert_indices_add, wte, learned_embedding_1, learned_embedding_2):
  B, S = bert_indices_add.shape
  t = B // 3
  nb = 3 * t
  n_tok = learned_embedding_1.shape[0]
  d = wte.shape[1]
  idx = bert_indices_add.astype(jnp.int32)
  shifted = jnp.pad(idx[t:nb, : S - n_tok], ((0, 0), (0, n_tok)))
  flat_idx = jnp.concatenate([idx[:t], shifted], axis=0).reshape(-1)
  pad = jnp.zeros((16 - n_tok, d), jnp.float32)
  learned = jnp.concatenate(
      [learned_embedding_1, pad, learned_embedding_2, pad], axis=0)
  out = _make_gather(nb, S, d, t)(flat_idx, wte, learned)
  return out.reshape(nb, S, d)


# speedup vs baseline: 4.0178x; 1.0721x over previous
import functools

import jax
import jax.numpy as jnp
from jax import lax
from jax.experimental import pallas as pl
from jax.experimental.pallas import tpu as pltpu
from jax.experimental.pallas import tpu_sc as plsc

N_TOK = 10
_info = plsc.get_sparse_core_info()
_NC = _info.num_cores
_NS = _info.num_subcores
_NW = _NC * _NS


def _make_gather(nb, seq, d, t):
  mesh = plsc.VectorSubcoreMesh(core_axis_name="c", subcore_axis_name="s")
  iters = (nb + _NW - 1) // _NW
  g0 = 128
  g1 = seq - g0
  g1p = seq - N_TOK - g0

  @functools.partial(
      pl.kernel,
      mesh=mesh,
      out_type=jax.ShapeDtypeStruct((nb * seq, d), jnp.float32),
      scratch_types=[
          pltpu.VMEM((seq,), jnp.int32),
          pltpu.VMEM((seq,), jnp.int32),
          pltpu.VMEM((N_TOK + seq, d), jnp.float32),
          pltpu.VMEM((N_TOK + seq, d), jnp.float32),
          pltpu.SemaphoreType.DMA,
          pltpu.SemaphoreType.DMA,
          pltpu.SemaphoreType.DMA,
          pltpu.SemaphoreType.DMA,
          pltpu.SemaphoreType.DMA,
      ],
  )
  def k(idx_hbm, table_hbm, learned_hbm, out_hbm, ix0, ix1, rv0, rv1, sg,
        so0, so1, si0, si1):
    wid = lax.axis_index("s") * _NC + lax.axis_index("c")
    idx_v = (ix0, ix1)
    rows_v = (rv0, rv1)
    sem_o = (so0, so1)
    sem_i = (si0, si1)

    def b_of(i):
      raw = wid + i * _NW
      return jnp.where(raw < nb, raw, wid)

    def idx_copy(p, b):
      return pltpu.make_async_copy(
          idx_hbm.at[pl.ds(b * seq, seq)], idx_v[p], sem_i[p])

    def out_wait(p):
      pltpu.make_async_copy(
          rows_v[p].at[pl.ds(0, seq)],
          out_hbm.at[pl.ds(b_of(0) * seq, seq)], sem_o[p]).wait()

    def step(i, p, prime=False):
      b = b_of(i)
      base = b * seq
      idx_copy(p, b).wait()
      idx_copy(1 - p, b_of(i + 1)).start()
      if not prime:
        out_wait(p)

      @pl.when((b >= t) & (b < t + 2 * _NW))
      def _():
        pltpu.sync_copy(
            learned_hbm.at[pl.ds(0, 16)], rows_v[p].at[pl.ds(0, 16)])

      @pl.when((b >= 2 * t) & (b < 2 * t + 2 * _NW))
      def _():
        pltpu.sync_copy(
            learned_hbm.at[pl.ds(16, 16)], rows_v[p].at[pl.ds(0, 16)])

      @pl.when(b < t)
      def _():
        c0 = pltpu.async_copy(
            table_hbm.at[idx_v[p].at[pl.ds(0, g0)]],
            rows_v[p].at[pl.ds(N_TOK, g0)], sg)
        c1 = pltpu.async_copy(
            table_hbm.at[idx_v[p].at[pl.ds(g0, g1)]],
            rows_v[p].at[pl.ds(N_TOK + g0, g1)], sg)
        c0.wait()
        c1.wait()
        pltpu.make_async_copy(
            rows_v[p].at[pl.ds(N_TOK, seq)],
            out_hbm.at[pl.ds(base, seq)], sem_o[p]).start()

      @pl.when(b >= t)
      def _():
        c0 = pltpu.async_copy(
            table_hbm.at[idx_v[p].at[pl.ds(0, g0)]],
            rows_v[p].at[pl.ds(N_TOK, g0)], sg)
        c1 = pltpu.async_copy(
            table_hbm.at[idx_v[p].at[pl.ds(g0, g1p)]],
            rows_v[p].at[pl.ds(N_TOK + g0, g1p)], sg)
        c0.wait()
        c1.wait()
        pltpu.make_async_copy(
            rows_v[p].at[pl.ds(0, seq)],
            out_hbm.at[pl.ds(base, seq)], sem_o[p]).start()

    idx_copy(0, b_of(0)).start()
    step(0, 0, prime=True)
    step(1, 1, prime=True)

    def body(j, _):
      step(2 * j, 0)
      step(2 * j + 1, 1)
      return 0

    lax.fori_loop(1, iters // 2, body, 0)

    idx_copy(0, b_of(iters)).wait()
    out_wait(0)
    out_wait(1)

  return k


def kernel(bert_indices_add, wte, learned_embedding_1, learned_embedding_2):
  B, S = bert_indices_add.shape
  t = B // 3
  nb = 3 * t
  n_tok = learned_embedding_1.shape[0]
  d = wte.shape[1]
  idx = bert_indices_add.astype(jnp.int32)
  shifted = jnp.pad(idx[t:nb, : S - n_tok], ((0, 0), (0, n_tok)))
  flat_idx = jnp.concatenate([idx[:t], shifted], axis=0).reshape(-1)
  pad = jnp.zeros((16 - n_tok, d), jnp.float32)
  learned = jnp.concatenate(
      [learned_embedding_1, pad, learned_embedding_2, pad], axis=0)
  out = _make_gather(nb, S, d, t)(flat_idx, wte, learned)
  return out.reshape(nb, S, d)
